# baseline (speedup 1.0000x reference)
import functools

import jax
import jax.numpy as jnp
from jax import lax
from jax.experimental import pallas as pl
from jax.experimental.pallas import tpu as pltpu
from jax.experimental.pallas import tpu_sc as plsc

N = 10000
E = 320000
D = 128
G = 16
STEPS = 2
NPAD = 10240
BLK = 2000

SC_CORES = 2
SC_TILES = 16
NW = SC_CORES * SC_TILES
ROWS_PER_TILE = NPAD // SC_TILES
UNIT = 128
NUNITS = E // UNIT
NU = NUNITS // NW
NEXTRA = NUNITS - NU * NW
WIN = 40

_sc_mesh = plsc.VectorSubcoreMesh(core_axis_name="c", subcore_axis_name="s")


def _inv_body(degp_ref, o_ref):
    d = degp_ref[0] + degp_ref[1] + 1.0
    o_ref[...] = lax.rsqrt(jnp.maximum(d, 1.0))


def _tc_inv(degp):
    return pl.pallas_call(
        _inv_body,
        out_shape=jax.ShapeDtypeStruct((2, NPAD), jnp.float32),
    )(degp)


def _embed_mlp_body(nodes_ref, we_ref, be_ref, w0_ref, b0_ref, inv_ref,
                    x_ref, h_ref):
    x = (
        jnp.dot(nodes_ref[...], we_ref[...], preferred_element_type=jnp.float32)
        + be_ref[...]
    )
    x_ref[...] = x
    h = jnp.dot(x, w0_ref[...], preferred_element_type=jnp.float32)
    h_ref[...] = jnp.maximum(h + b0_ref[...], 0.0) * inv_ref[...]


def _tc_embed_mlp(nodes, we, be, w0, b0, inv_s_col):
    return pl.pallas_call(
        _embed_mlp_body,
        grid=(N // BLK,),
        in_specs=[
            pl.BlockSpec((BLK, D), lambda i: (i, 0)),
            pl.BlockSpec((D, D), lambda i: (0, 0)),
            pl.BlockSpec((1, D), lambda i: (0, 0)),
            pl.BlockSpec((D, D), lambda i: (0, 0)),
            pl.BlockSpec((1, D), lambda i: (0, 0)),
            pl.BlockSpec((BLK, 1), lambda i: (i, 0)),
        ],
        out_specs=[
            pl.BlockSpec((BLK, D), lambda i: (i, 0)),
            pl.BlockSpec((BLK, D), lambda i: (i, 0)),
        ],
        out_shape=[
            jax.ShapeDtypeStruct((N, D), jnp.float32),
            jax.ShapeDtypeStruct((N, D), jnp.float32),
        ],
    )(nodes, we, be.reshape(1, D), w0, b0.reshape(1, D), inv_s_col)


def _ln(t, s, b):
    mu = jnp.mean(t, axis=-1, keepdims=True)
    var = jnp.mean(jnp.square(t - mu), axis=-1, keepdims=True)
    return (t - mu) * lax.rsqrt(var + 1e-6) * s + b


def _post_mlp_body(aggp_ref, h_ref, x_ref, invr_ref, s_ref, b_ref,
                   w1_ref, b1_ref, invs_ref, x_out, h_out):
    agg = aggp_ref[0] + aggp_ref[1] + h_ref[...]
    t = agg * invr_ref[...] + x_ref[...]
    xn = _ln(t, s_ref[...], b_ref[...])
    x_out[...] = xn
    h2 = jnp.dot(xn, w1_ref[...], preferred_element_type=jnp.float32)
    h_out[...] = jnp.maximum(h2 + b1_ref[...], 0.0) * invs_ref[...]


def _tc_post_mlp(aggp, h, x, inv_r_col, scale, bias, w1, b1, inv_s_col):
    return pl.pallas_call(
        _post_mlp_body,
        grid=(N // BLK,),
        in_specs=[
            pl.BlockSpec((2, BLK, D), lambda i: (0, i, 0)),
            pl.BlockSpec((BLK, D), lambda i: (i, 0)),
            pl.BlockSpec((BLK, D), lambda i: (i, 0)),
            pl.BlockSpec((BLK, 1), lambda i: (i, 0)),
            pl.BlockSpec((1, D), lambda i: (0, 0)),
            pl.BlockSpec((1, D), lambda i: (0, 0)),
            pl.BlockSpec((D, D), lambda i: (0, 0)),
            pl.BlockSpec((1, D), lambda i: (0, 0)),
            pl.BlockSpec((BLK, 1), lambda i: (i, 0)),
        ],
        out_specs=[
            pl.BlockSpec((BLK, D), lambda i: (i, 0)),
            pl.BlockSpec((BLK, D), lambda i: (i, 0)),
        ],
        out_shape=[
            jax.ShapeDtypeStruct((N, D), jnp.float32),
            jax.ShapeDtypeStruct((N, D), jnp.float32),
        ],
    )(aggp, h, x, inv_r_col, scale.reshape(1, D), bias.reshape(1, D),
      w1, b1.reshape(1, D), inv_s_col)


def _post_pool_body(aggp_ref, h_ref, x_ref, invr_ref, s_ref, b_ref,
                    p_ref, cnt_ref, wd_ref, bd_ref, o_ref, acc_ref):
    i = pl.program_id(0)
    agg = aggp_ref[0] + aggp_ref[1] + h_ref[...]
    t = agg * invr_ref[...] + x_ref[...]
    xn = _ln(t, s_ref[...], b_ref[...])
    partial = lax.dot_general(
        p_ref[...], xn, (((0,), (0,)), ((), ())),
        preferred_element_type=jnp.float32,
    )

    @pl.when(i == 0)
    def _():
        acc_ref[...] = partial

    @pl.when(i > 0)
    def _():
        acc_ref[...] = acc_ref[...] + partial

    @pl.when(i == N // BLK - 1)
    def _():
        pooled = acc_ref[...] / cnt_ref[...]
        o_ref[...] = (
            jnp.dot(pooled, wd_ref[...], preferred_element_type=jnp.float32)
            + bd_ref[...]
        )


def _tc_post_pool(aggp, h, x, inv_r_col, scale, bias, n_node, w_dec, b_dec):
    bounds = jnp.cumsum(n_node)
    node_graph = jnp.sum(
        jnp.arange(N, dtype=jnp.int32)[:, None] >= bounds[None, :], axis=1
    )
    p = (node_graph[:, None] == jnp.arange(G, dtype=jnp.int32)[None, :])
    p = p.astype(jnp.float32)
    counts = jnp.maximum(n_node.astype(jnp.float32), 1.0).reshape(G, 1)
    return pl.pallas_call(
        _post_pool_body,
        grid=(N // BLK,),
        in_specs=[
            pl.BlockSpec((2, BLK, D), lambda i: (0, i, 0)),
            pl.BlockSpec((BLK, D), lambda i: (i, 0)),
            pl.BlockSpec((BLK, D), lambda i: (i, 0)),
            pl.BlockSpec((BLK, 1), lambda i: (i, 0)),
            pl.BlockSpec((1, D), lambda i: (0, 0)),
            pl.BlockSpec((1, D), lambda i: (0, 0)),
            pl.BlockSpec((BLK, G), lambda i: (i, 0)),
            pl.BlockSpec((G, 1), lambda i: (0, 0)),
            pl.BlockSpec((D, D), lambda i: (0, 0)),
            pl.BlockSpec((1, D), lambda i: (0, 0)),
        ],
        out_specs=pl.BlockSpec((G, D), lambda i: (0, 0)),
        out_shape=jax.ShapeDtypeStruct((G, D), jnp.float32),
        scratch_shapes=[pltpu.VMEM((G, D), jnp.float32)],
    )(aggp, h, x, inv_r_col, scale.reshape(1, D), bias.reshape(1, D),
      p, counts, w_dec, b_dec.reshape(1, D))


def _unit_range(wid):
    start = NU * wid + jnp.maximum(wid - (NW - NEXTRA), 0)
    cnt = NU + (wid >= (NW - NEXTRA)).astype(jnp.int32)
    return start, cnt


@functools.partial(
    pl.kernel,
    out_type=jax.ShapeDtypeStruct((SC_CORES, 2, NPAD), jnp.float32),
    mesh=_sc_mesh,
    scratch_types=[
        pltpu.VMEM((NU + 1, 2, UNIT), jnp.int32),
        pltpu.VMEM((UNIT,), jnp.float32),
        pltpu.VMEM((ROWS_PER_TILE,), jnp.float32),
        pltpu.VMEM_SHARED((NPAD,), jnp.float32),
        pltpu.VMEM_SHARED((NPAD,), jnp.float32),
    ],
)
def _sc_degrees(comb_hbm, out_hbm, cbuf, ones_v, tmp, acc_s, acc_r):
    cid = lax.axis_index("c")
    sid = lax.axis_index("s")
    wid = sid * SC_CORES + cid
    start, cnt = _unit_range(wid)
    pltpu.sync_copy(comb_hbm.at[pl.ds(start, NU + 1)], cbuf)

    for k in range(UNIT // 16):
        ones_v[pl.ds(k * 16, 16)] = jnp.ones((16,), jnp.float32)
    for k in range(ROWS_PER_TILE // 16):
        tmp[pl.ds(k * 16, 16)] = jnp.zeros((16,), jnp.float32)
    base_r = sid * ROWS_PER_TILE
    pltpu.sync_copy(tmp, acc_s.at[pl.ds(base_r, ROWS_PER_TILE)])
    pltpu.sync_copy(tmp, acc_r.at[pl.ds(base_r, ROWS_PER_TILE)])
    plsc.subcore_barrier()

    def body(j, carry):
        pltpu.sync_copy(ones_v, acc_s.at[cbuf.at[j, 0]], add=True)
        pltpu.sync_copy(ones_v, acc_r.at[cbuf.at[j, 1]], add=True)
        return carry

    lax.fori_loop(0, cnt, body, 0)
    plsc.subcore_barrier()
    pltpu.sync_copy(acc_s.at[pl.ds(base_r, ROWS_PER_TILE)], tmp)
    pltpu.sync_copy(tmp, out_hbm.at[cid, 0, pl.ds(base_r, ROWS_PER_TILE)])
    pltpu.sync_copy(acc_r.at[pl.ds(base_r, ROWS_PER_TILE)], tmp)
    pltpu.sync_copy(tmp, out_hbm.at[cid, 1, pl.ds(base_r, ROWS_PER_TILE)])


@functools.partial(
    pl.kernel,
    out_type=jax.ShapeDtypeStruct((SC_CORES, NPAD, D), jnp.float32),
    mesh=_sc_mesh,
    scratch_types=[
        pltpu.VMEM((WIN, 2, UNIT), jnp.int32),
        pltpu.VMEM((UNIT, D), jnp.float32),
        pltpu.VMEM((UNIT, D), jnp.float32),
        pltpu.VMEM_SHARED((NPAD, D), jnp.float32),
        pltpu.SemaphoreType.DMA,
        pltpu.SemaphoreType.DMA,
        pltpu.SemaphoreType.DMA,
        pltpu.SemaphoreType.DMA,
    ],
)
def _sc_edge_agg(h_hbm, comb_hbm, out_hbm, cbuf, rows0, rows1, acc,
                 semg0, semg1, sems0, sems1):
    cid = lax.axis_index("c")
    sid = lax.axis_index("s")
    wid = sid * SC_CORES + cid
    start, cnt = _unit_range(wid)

    def zrow(r, carry):
        for k in range(D // 16):
            rows0[r, pl.ds(k * 16, 16)] = jnp.zeros((16,), jnp.float32)
        return carry

    lax.fori_loop(0, UNIT, zrow, 0)
    base_r = sid * ROWS_PER_TILE
    for k in range(ROWS_PER_TILE // UNIT):
        pltpu.sync_copy(rows0, acc.at[pl.ds(base_r + k * UNIT, UNIT)])
    plsc.subcore_barrier()

    def window(w0, wcnt):
        pltpu.sync_copy(comb_hbm.at[pl.ds(w0, WIN)], cbuf)
        pltpu.async_copy(h_hbm.at[cbuf.at[0, 0]], rows0, semg0)
        pltpu.async_copy(h_hbm.at[cbuf.at[1, 0]], rows1, semg1)

        def body(p, carry):
            u0 = 2 * p
            pltpu.make_async_copy(h_hbm.at[cbuf.at[u0, 0]], rows0,
                                  semg0).wait()
            pltpu.async_copy(rows0, acc.at[cbuf.at[u0, 1]], sems0, add=True)

            @pl.when(u0 + 1 < wcnt)
            def _():
                pltpu.make_async_copy(h_hbm.at[cbuf.at[u0 + 1, 0]], rows1,
                                      semg1).wait()
                pltpu.async_copy(rows1, acc.at[cbuf.at[u0 + 1, 1]], sems1,
                                 add=True)

            @pl.when(u0 + 2 < wcnt)
            def _():
                pltpu.make_async_copy(rows0, acc.at[cbuf.at[u0, 1]],
                                      sems0).wait()
                pltpu.async_copy(h_hbm.at[cbuf.at[u0 + 2, 0]], rows0, semg0)

            @pl.when(u0 + 3 < wcnt)
            def _():
                pltpu.make_async_copy(rows1, acc.at[cbuf.at[u0 + 1, 1]],
                                      sems1).wait()
                pltpu.async_copy(h_hbm.at[cbuf.at[u0 + 3, 0]], rows1, semg1)

            return carry

        lax.fori_loop(0, (wcnt + 1) // 2, body, 0)
        pltpu.make_async_copy(rows0, acc.at[cbuf.at[0, 1]], sems0).wait()
        pltpu.make_async_copy(rows1, acc.at[cbuf.at[0, 1]], sems1).wait()

    window(start, jnp.int32(WIN))
    window(start + WIN, cnt - WIN)

    plsc.subcore_barrier()
    for k in range(ROWS_PER_TILE // UNIT):
        r0 = base_r + k * UNIT
        pltpu.sync_copy(acc.at[pl.ds(r0, UNIT)], rows0)
        pltpu.sync_copy(rows0, out_hbm.at[cid, pl.ds(r0, UNIT)])


def kernel(nodes, senders, receivers, n_node, W_embed, b_embed, W_mlp, b_mlp,
           ln_scale, ln_bias, W_dec, b_dec):
    comb = jnp.stack(
        [senders.reshape(NUNITS, UNIT), receivers.reshape(NUNITS, UNIT)],
        axis=1,
    )
    comb = jnp.pad(comb, ((0, 4), (0, 0), (0, 0)))
    degp = _sc_degrees(comb)
    inv = _tc_inv(degp)
    inv_s_col = inv[0].reshape(NPAD, 1)
    inv_r_col = inv[1].reshape(NPAD, 1)
    x, h = _tc_embed_mlp(nodes, W_embed, b_embed, W_mlp[0], b_mlp[0],
                         inv_s_col)
    aggp = _sc_edge_agg(h, comb)
    x, h = _tc_post_mlp(aggp, h, x, inv_r_col, ln_scale[0], ln_bias[0],
                        W_mlp[1], b_mlp[1], inv_s_col)
    aggp = _sc_edge_agg(h, comb)
    return _tc_post_pool(aggp, h, x, inv_r_col, ln_scale[1], ln_bias[1],
                         n_node, W_dec, b_dec)


# speedup vs baseline: 1.2304x; 1.2304x over previous
import functools

import jax
import jax.numpy as jnp
from jax import lax
from jax.experimental import pallas as pl
from jax.experimental.pallas import tpu as pltpu
from jax.experimental.pallas import tpu_sc as plsc

N = 10000
E = 320000
D = 128
G = 16
STEPS = 2
NPAD = 10240
BLK = 2000

SC_CORES = 2
SC_TILES = 16
NW = SC_CORES * SC_TILES
ROWS_PER_TILE = NPAD // SC_TILES
UNIT = 128
NUNITS = E // UNIT
NU = NUNITS // NW
NEXTRA = NUNITS - NU * NW
WIN = 40

_sc_mesh = plsc.VectorSubcoreMesh(core_axis_name="c", subcore_axis_name="s")


def _inv_body(degp_ref, o_ref):
    d = degp_ref[0] + degp_ref[1] + 1.0
    o_ref[...] = lax.rsqrt(jnp.maximum(d, 1.0))


def _tc_inv(degp):
    return pl.pallas_call(
        _inv_body,
        out_shape=jax.ShapeDtypeStruct((2, NPAD), jnp.float32),
    )(degp)


def _embed_mlp_body(nodes_ref, we_ref, be_ref, w0_ref, b0_ref, inv_ref,
                    x_ref, h_ref):
    x = (
        jnp.dot(nodes_ref[...], we_ref[...], preferred_element_type=jnp.float32)
        + be_ref[...]
    )
    x_ref[...] = x
    h = jnp.dot(x, w0_ref[...], preferred_element_type=jnp.float32)
    h_ref[...] = jnp.maximum(h + b0_ref[...], 0.0) * inv_ref[...]


def _tc_embed_mlp(nodes, we, be, w0, b0, inv_s_col):
    return pl.pallas_call(
        _embed_mlp_body,
        grid=(N // BLK,),
        in_specs=[
            pl.BlockSpec((BLK, D), lambda i: (i, 0)),
            pl.BlockSpec((D, D), lambda i: (0, 0)),
            pl.BlockSpec((1, D), lambda i: (0, 0)),
            pl.BlockSpec((D, D), lambda i: (0, 0)),
            pl.BlockSpec((1, D), lambda i: (0, 0)),
            pl.BlockSpec((BLK, 1), lambda i: (i, 0)),
        ],
        out_specs=[
            pl.BlockSpec((BLK, D), lambda i: (i, 0)),
            pl.BlockSpec((BLK, D), lambda i: (i, 0)),
        ],
        out_shape=[
            jax.ShapeDtypeStruct((N, D), jnp.float32),
            jax.ShapeDtypeStruct((N, D), jnp.float32),
        ],
    )(nodes, we, be.reshape(1, D), w0, b0.reshape(1, D), inv_s_col)


def _ln(t, s, b):
    mu = jnp.mean(t, axis=-1, keepdims=True)
    var = jnp.mean(jnp.square(t - mu), axis=-1, keepdims=True)
    return (t - mu) * lax.rsqrt(var + 1e-6) * s + b


def _post_mlp_body(aggp_ref, h_ref, x_ref, invr_ref, s_ref, b_ref,
                   w1_ref, b1_ref, invs_ref, x_out, h_out):
    agg = aggp_ref[0] + aggp_ref[1] + h_ref[...]
    t = agg * invr_ref[...] + x_ref[...]
    xn = _ln(t, s_ref[...], b_ref[...])
    x_out[...] = xn
    h2 = jnp.dot(xn, w1_ref[...], preferred_element_type=jnp.float32)
    h_out[...] = jnp.maximum(h2 + b1_ref[...], 0.0) * invs_ref[...]


def _tc_post_mlp(aggp, h, x, inv_r_col, scale, bias, w1, b1, inv_s_col):
    return pl.pallas_call(
        _post_mlp_body,
        grid=(N // BLK,),
        in_specs=[
            pl.BlockSpec((2, BLK, D), lambda i: (0, i, 0)),
            pl.BlockSpec((BLK, D), lambda i: (i, 0)),
            pl.BlockSpec((BLK, D), lambda i: (i, 0)),
            pl.BlockSpec((BLK, 1), lambda i: (i, 0)),
            pl.BlockSpec((1, D), lambda i: (0, 0)),
            pl.BlockSpec((1, D), lambda i: (0, 0)),
            pl.BlockSpec((D, D), lambda i: (0, 0)),
            pl.BlockSpec((1, D), lambda i: (0, 0)),
            pl.BlockSpec((BLK, 1), lambda i: (i, 0)),
        ],
        out_specs=[
            pl.BlockSpec((BLK, D), lambda i: (i, 0)),
            pl.BlockSpec((BLK, D), lambda i: (i, 0)),
        ],
        out_shape=[
            jax.ShapeDtypeStruct((N, D), jnp.float32),
            jax.ShapeDtypeStruct((N, D), jnp.float32),
        ],
    )(aggp, h, x, inv_r_col, scale.reshape(1, D), bias.reshape(1, D),
      w1, b1.reshape(1, D), inv_s_col)


def _post_pool_body(aggp_ref, h_ref, x_ref, invr_ref, s_ref, b_ref,
                    p_ref, cnt_ref, wd_ref, bd_ref, o_ref, acc_ref):
    i = pl.program_id(0)
    agg = aggp_ref[0] + aggp_ref[1] + h_ref[...]
    t = agg * invr_ref[...] + x_ref[...]
    xn = _ln(t, s_ref[...], b_ref[...])
    partial = lax.dot_general(
        p_ref[...], xn, (((0,), (0,)), ((), ())),
        preferred_element_type=jnp.float32,
    )

    @pl.when(i == 0)
    def _():
        acc_ref[...] = partial

    @pl.when(i > 0)
    def _():
        acc_ref[...] = acc_ref[...] + partial

    @pl.when(i == N // BLK - 1)
    def _():
        pooled = acc_ref[...] / cnt_ref[...]
        o_ref[...] = (
            jnp.dot(pooled, wd_ref[...], preferred_element_type=jnp.float32)
            + bd_ref[...]
        )


def _tc_post_pool(aggp, h, x, inv_r_col, scale, bias, n_node, w_dec, b_dec):
    bounds = jnp.cumsum(n_node)
    node_graph = jnp.sum(
        jnp.arange(N, dtype=jnp.int32)[:, None] >= bounds[None, :], axis=1
    )
    p = (node_graph[:, None] == jnp.arange(G, dtype=jnp.int32)[None, :])
    p = p.astype(jnp.float32)
    counts = jnp.maximum(n_node.astype(jnp.float32), 1.0).reshape(G, 1)
    return pl.pallas_call(
        _post_pool_body,
        grid=(N // BLK,),
        in_specs=[
            pl.BlockSpec((2, BLK, D), lambda i: (0, i, 0)),
            pl.BlockSpec((BLK, D), lambda i: (i, 0)),
            pl.BlockSpec((BLK, D), lambda i: (i, 0)),
            pl.BlockSpec((BLK, 1), lambda i: (i, 0)),
            pl.BlockSpec((1, D), lambda i: (0, 0)),
            pl.BlockSpec((1, D), lambda i: (0, 0)),
            pl.BlockSpec((BLK, G), lambda i: (i, 0)),
            pl.BlockSpec((G, 1), lambda i: (0, 0)),
            pl.BlockSpec((D, D), lambda i: (0, 0)),
            pl.BlockSpec((1, D), lambda i: (0, 0)),
        ],
        out_specs=pl.BlockSpec((G, D), lambda i: (0, 0)),
        out_shape=jax.ShapeDtypeStruct((G, D), jnp.float32),
        scratch_shapes=[pltpu.VMEM((G, D), jnp.float32)],
    )(aggp, h, x, inv_r_col, scale.reshape(1, D), bias.reshape(1, D),
      p, counts, w_dec, b_dec.reshape(1, D))


def _unit_range(wid):
    start = NU * wid + jnp.maximum(wid - (NW - NEXTRA), 0)
    cnt = NU + (wid >= (NW - NEXTRA)).astype(jnp.int32)
    return start, cnt


@functools.partial(
    pl.kernel,
    out_type=jax.ShapeDtypeStruct((SC_CORES, 2, NPAD), jnp.float32),
    mesh=_sc_mesh,
    scratch_types=[
        pltpu.VMEM((NU + 1, 2, UNIT), jnp.int32),
        pltpu.VMEM((UNIT,), jnp.float32),
        pltpu.VMEM((ROWS_PER_TILE,), jnp.float32),
        pltpu.VMEM_SHARED((NPAD,), jnp.float32),
        pltpu.VMEM_SHARED((NPAD,), jnp.float32),
    ],
)
def _sc_degrees(comb_hbm, out_hbm, cbuf, ones_v, tmp, acc_s, acc_r):
    cid = lax.axis_index("c")
    sid = lax.axis_index("s")
    wid = sid * SC_CORES + cid
    start, cnt = _unit_range(wid)
    pltpu.sync_copy(comb_hbm.at[pl.ds(start, NU + 1)], cbuf)

    for k in range(UNIT // 16):
        ones_v[pl.ds(k * 16, 16)] = jnp.ones((16,), jnp.float32)
    for k in range(ROWS_PER_TILE // 16):
        tmp[pl.ds(k * 16, 16)] = jnp.zeros((16,), jnp.float32)
    base_r = sid * ROWS_PER_TILE
    pltpu.sync_copy(tmp, acc_s.at[pl.ds(base_r, ROWS_PER_TILE)])
    pltpu.sync_copy(tmp, acc_r.at[pl.ds(base_r, ROWS_PER_TILE)])
    plsc.subcore_barrier()

    def body(j, carry):
        pltpu.sync_copy(ones_v, acc_s.at[cbuf.at[j, 0]], add=True)
        pltpu.sync_copy(ones_v, acc_r.at[cbuf.at[j, 1]], add=True)
        return carry

    lax.fori_loop(0, cnt, body, 0)
    plsc.subcore_barrier()
    pltpu.sync_copy(acc_s.at[pl.ds(base_r, ROWS_PER_TILE)], tmp)
    pltpu.sync_copy(tmp, out_hbm.at[cid, 0, pl.ds(base_r, ROWS_PER_TILE)])
    pltpu.sync_copy(acc_r.at[pl.ds(base_r, ROWS_PER_TILE)], tmp)
    pltpu.sync_copy(tmp, out_hbm.at[cid, 1, pl.ds(base_r, ROWS_PER_TILE)])


@functools.partial(
    pl.kernel,
    out_type=jax.ShapeDtypeStruct((SC_CORES, NPAD, D), jnp.float32),
    mesh=_sc_mesh,
    scratch_types=[
        pltpu.VMEM((WIN, 2, UNIT), jnp.int32),
        pltpu.VMEM((UNIT, D), jnp.float32),
        pltpu.VMEM((UNIT, D), jnp.float32),
        pltpu.VMEM_SHARED((NPAD, D), jnp.float32),
        pltpu.SemaphoreType.DMA,
        pltpu.SemaphoreType.DMA,
    ],
)
def _sc_edge_agg(h_hbm, comb_hbm, out_hbm, cbuf, rows0, rows1, acc,
                 sem0, sem1):
    cid = lax.axis_index("c")
    sid = lax.axis_index("s")
    wid = sid * SC_CORES + cid
    start, cnt = _unit_range(wid)

    def zrow(r, carry):
        for k in range(D // 16):
            rows0[r, pl.ds(k * 16, 16)] = jnp.zeros((16,), jnp.float32)
        return carry

    lax.fori_loop(0, UNIT, zrow, 0)
    base_r = sid * ROWS_PER_TILE
    for k in range(ROWS_PER_TILE // UNIT):
        pltpu.sync_copy(rows0, acc.at[pl.ds(base_r + k * UNIT, UNIT)])
    plsc.subcore_barrier()

    def window(w0, wcnt):
        pltpu.sync_copy(comb_hbm.at[pl.ds(w0, WIN)], cbuf)
        pltpu.async_copy(h_hbm.at[cbuf.at[0, 0]], rows0, sem0)

        def body(p, carry):
            u0 = 2 * p

            @pl.when(u0 + 1 < wcnt)
            def _():
                pltpu.async_copy(h_hbm.at[cbuf.at[u0 + 1, 0]], rows1, sem1)

            pltpu.make_async_copy(h_hbm.at[cbuf.at[u0, 0]], rows0,
                                  sem0).wait()
            pltpu.sync_copy(rows0, acc.at[cbuf.at[u0, 1]], add=True)

            @pl.when(u0 + 2 < wcnt)
            def _():
                pltpu.async_copy(h_hbm.at[cbuf.at[u0 + 2, 0]], rows0, sem0)

            @pl.when(u0 + 1 < wcnt)
            def _():
                pltpu.make_async_copy(h_hbm.at[cbuf.at[u0 + 1, 0]], rows1,
                                      sem1).wait()
                pltpu.sync_copy(rows1, acc.at[cbuf.at[u0 + 1, 1]], add=True)

            return carry

        lax.fori_loop(0, (wcnt + 1) // 2, body, 0)

    window(start, jnp.int32(WIN))
    window(start + WIN, cnt - WIN)

    plsc.subcore_barrier()
    for k in range(ROWS_PER_TILE // UNIT):
        r0 = base_r + k * UNIT
        pltpu.sync_copy(acc.at[pl.ds(r0, UNIT)], rows0)
        pltpu.sync_copy(rows0, out_hbm.at[cid, pl.ds(r0, UNIT)])


def kernel(nodes, senders, receivers, n_node, W_embed, b_embed, W_mlp, b_mlp,
           ln_scale, ln_bias, W_dec, b_dec):
    comb = jnp.stack(
        [senders.reshape(NUNITS, UNIT), receivers.reshape(NUNITS, UNIT)],
        axis=1,
    )
    comb = jnp.pad(comb, ((0, 4), (0, 0), (0, 0)))
    degp = _sc_degrees(comb)
    inv = _tc_inv(degp)
    inv_s_col = inv[0].reshape(NPAD, 1)
    inv_r_col = inv[1].reshape(NPAD, 1)
    x, h = _tc_embed_mlp(nodes, W_embed, b_embed, W_mlp[0], b_mlp[0],
                         inv_s_col)
    aggp = _sc_edge_agg(h, comb)
    x, h = _tc_post_mlp(aggp, h, x, inv_r_col, ln_scale[0], ln_bias[0],
                        W_mlp[1], b_mlp[1], inv_s_col)
    aggp = _sc_edge_agg(h, comb)
    return _tc_post_pool(aggp, h, x, inv_r_col, ln_scale[1], ln_bias[1],
                         n_node, W_dec, b_dec)


# speedup vs baseline: 1.2753x; 1.0365x over previous
import functools

import jax
import jax.numpy as jnp
from jax import lax
from jax.experimental import pallas as pl
from jax.experimental.pallas import tpu as pltpu
from jax.experimental.pallas import tpu_sc as plsc

N = 10000
E = 320000
D = 128
G = 16
STEPS = 2
NPAD = 10240
BLK = 2000

SC_CORES = 2
SC_TILES = 16
NW = SC_CORES * SC_TILES
ROWS_PER_TILE = NPAD // SC_TILES
UNIT = 128
NUNITS = E // UNIT
NU = NUNITS // NW
NEXTRA = NUNITS - NU * NW
WIN = 60

_sc_mesh = plsc.VectorSubcoreMesh(core_axis_name="c", subcore_axis_name="s")


def _inv_body(degp_ref, o_ref):
    d = degp_ref[0] + degp_ref[1] + 1.0
    o_ref[...] = lax.rsqrt(jnp.maximum(d, 1.0))


def _tc_inv(degp):
    return pl.pallas_call(
        _inv_body,
        out_shape=jax.ShapeDtypeStruct((2, NPAD), jnp.float32),
    )(degp)


def _embed_mlp_body(nodes_ref, we_ref, be_ref, w0_ref, b0_ref, inv_ref,
                    x_ref, h_ref):
    x = (
        jnp.dot(nodes_ref[...], we_ref[...], preferred_element_type=jnp.float32)
        + be_ref[...]
    )
    x_ref[...] = x
    h = jnp.dot(x, w0_ref[...], preferred_element_type=jnp.float32)
    h_ref[...] = jnp.maximum(h + b0_ref[...], 0.0) * inv_ref[...]


def _tc_embed_mlp(nodes, we, be, w0, b0, inv_s_col):
    return pl.pallas_call(
        _embed_mlp_body,
        grid=(N // BLK,),
        in_specs=[
            pl.BlockSpec((BLK, D), lambda i: (i, 0)),
            pl.BlockSpec((D, D), lambda i: (0, 0)),
            pl.BlockSpec((1, D), lambda i: (0, 0)),
            pl.BlockSpec((D, D), lambda i: (0, 0)),
            pl.BlockSpec((1, D), lambda i: (0, 0)),
            pl.BlockSpec((BLK, 1), lambda i: (i, 0)),
        ],
        out_specs=[
            pl.BlockSpec((BLK, D), lambda i: (i, 0)),
            pl.BlockSpec((BLK, D), lambda i: (i, 0)),
        ],
        out_shape=[
            jax.ShapeDtypeStruct((N, D), jnp.float32),
            jax.ShapeDtypeStruct((N, D), jnp.float32),
        ],
    )(nodes, we, be.reshape(1, D), w0, b0.reshape(1, D), inv_s_col)


def _ln(t, s, b):
    mu = jnp.mean(t, axis=-1, keepdims=True)
    var = jnp.mean(jnp.square(t - mu), axis=-1, keepdims=True)
    return (t - mu) * lax.rsqrt(var + 1e-6) * s + b


def _post_mlp_body(aggp_ref, h_ref, x_ref, invr_ref, s_ref, b_ref,
                   w1_ref, b1_ref, invs_ref, x_out, h_out):
    agg = aggp_ref[0] + aggp_ref[1] + h_ref[...]
    t = agg * invr_ref[...] + x_ref[...]
    xn = _ln(t, s_ref[...], b_ref[...])
    x_out[...] = xn
    h2 = jnp.dot(xn, w1_ref[...], preferred_element_type=jnp.float32)
    h_out[...] = jnp.maximum(h2 + b1_ref[...], 0.0) * invs_ref[...]


def _tc_post_mlp(aggp, h, x, inv_r_col, scale, bias, w1, b1, inv_s_col):
    return pl.pallas_call(
        _post_mlp_body,
        grid=(N // BLK,),
        in_specs=[
            pl.BlockSpec((2, BLK, D), lambda i: (0, i, 0)),
            pl.BlockSpec((BLK, D), lambda i: (i, 0)),
            pl.BlockSpec((BLK, D), lambda i: (i, 0)),
            pl.BlockSpec((BLK, 1), lambda i: (i, 0)),
            pl.BlockSpec((1, D), lambda i: (0, 0)),
            pl.BlockSpec((1, D), lambda i: (0, 0)),
            pl.BlockSpec((D, D), lambda i: (0, 0)),
            pl.BlockSpec((1, D), lambda i: (0, 0)),
            pl.BlockSpec((BLK, 1), lambda i: (i, 0)),
        ],
        out_specs=[
            pl.BlockSpec((BLK, D), lambda i: (i, 0)),
            pl.BlockSpec((BLK, D), lambda i: (i, 0)),
        ],
        out_shape=[
            jax.ShapeDtypeStruct((N, D), jnp.float32),
            jax.ShapeDtypeStruct((N, D), jnp.float32),
        ],
    )(aggp, h, x, inv_r_col, scale.reshape(1, D), bias.reshape(1, D),
      w1, b1.reshape(1, D), inv_s_col)


def _post_pool_body(aggp_ref, h_ref, x_ref, invr_ref, s_ref, b_ref,
                    p_ref, cnt_ref, wd_ref, bd_ref, o_ref, acc_ref):
    i = pl.program_id(0)
    agg = aggp_ref[0] + aggp_ref[1] + h_ref[...]
    t = agg * invr_ref[...] + x_ref[...]
    xn = _ln(t, s_ref[...], b_ref[...])
    partial = lax.dot_general(
        p_ref[...], xn, (((0,), (0,)), ((), ())),
        preferred_element_type=jnp.float32,
    )

    @pl.when(i == 0)
    def _():
        acc_ref[...] = partial

    @pl.when(i > 0)
    def _():
        acc_ref[...] = acc_ref[...] + partial

    @pl.when(i == N // BLK - 1)
    def _():
        pooled = acc_ref[...] / cnt_ref[...]
        o_ref[...] = (
            jnp.dot(pooled, wd_ref[...], preferred_element_type=jnp.float32)
            + bd_ref[...]
        )


def _tc_post_pool(aggp, h, x, inv_r_col, scale, bias, n_node, w_dec, b_dec):
    bounds = jnp.cumsum(n_node)
    node_graph = jnp.sum(
        jnp.arange(N, dtype=jnp.int32)[:, None] >= bounds[None, :], axis=1
    )
    p = (node_graph[:, None] == jnp.arange(G, dtype=jnp.int32)[None, :])
    p = p.astype(jnp.float32)
    counts = jnp.maximum(n_node.astype(jnp.float32), 1.0).reshape(G, 1)
    return pl.pallas_call(
        _post_pool_body,
        grid=(N // BLK,),
        in_specs=[
            pl.BlockSpec((2, BLK, D), lambda i: (0, i, 0)),
            pl.BlockSpec((BLK, D), lambda i: (i, 0)),
            pl.BlockSpec((BLK, D), lambda i: (i, 0)),
            pl.BlockSpec((BLK, 1), lambda i: (i, 0)),
            pl.BlockSpec((1, D), lambda i: (0, 0)),
            pl.BlockSpec((1, D), lambda i: (0, 0)),
            pl.BlockSpec((BLK, G), lambda i: (i, 0)),
            pl.BlockSpec((G, 1), lambda i: (0, 0)),
            pl.BlockSpec((D, D), lambda i: (0, 0)),
            pl.BlockSpec((1, D), lambda i: (0, 0)),
        ],
        out_specs=pl.BlockSpec((G, D), lambda i: (0, 0)),
        out_shape=jax.ShapeDtypeStruct((G, D), jnp.float32),
        scratch_shapes=[pltpu.VMEM((G, D), jnp.float32)],
    )(aggp, h, x, inv_r_col, scale.reshape(1, D), bias.reshape(1, D),
      p, counts, w_dec, b_dec.reshape(1, D))


def _unit_range(wid):
    start = NU * wid + jnp.maximum(wid - (NW - NEXTRA), 0)
    cnt = NU + (wid >= (NW - NEXTRA)).astype(jnp.int32)
    return start, cnt


@functools.partial(
    pl.kernel,
    out_type=jax.ShapeDtypeStruct((SC_CORES, 2, NPAD), jnp.float32),
    mesh=_sc_mesh,
    scratch_types=[
        pltpu.VMEM((NU + 1, 2, UNIT), jnp.int32),
        pltpu.VMEM((UNIT,), jnp.float32),
        pltpu.VMEM((ROWS_PER_TILE,), jnp.float32),
        pltpu.VMEM_SHARED((NPAD,), jnp.float32),
        pltpu.VMEM_SHARED((NPAD,), jnp.float32),
        pltpu.SemaphoreType.DMA,
    ],
)
def _sc_degrees(comb_hbm, out_hbm, cbuf, ones_v, tmp, acc_s, acc_r, sem):
    cid = lax.axis_index("c")
    sid = lax.axis_index("s")
    wid = sid * SC_CORES + cid
    start, cnt = _unit_range(wid)
    pltpu.sync_copy(comb_hbm.at[pl.ds(start, NU + 1)], cbuf)

    for k in range(UNIT // 16):
        ones_v[pl.ds(k * 16, 16)] = jnp.ones((16,), jnp.float32)
    for k in range(ROWS_PER_TILE // 16):
        tmp[pl.ds(k * 16, 16)] = jnp.zeros((16,), jnp.float32)
    base_r = sid * ROWS_PER_TILE
    pltpu.sync_copy(tmp, acc_s.at[pl.ds(base_r, ROWS_PER_TILE)])
    pltpu.sync_copy(tmp, acc_r.at[pl.ds(base_r, ROWS_PER_TILE)])
    plsc.subcore_barrier()

    def body(j, carry):
        pltpu.async_copy(ones_v, acc_s.at[cbuf.at[j, 0]], sem, add=True)
        pltpu.async_copy(ones_v, acc_r.at[cbuf.at[j, 1]], sem, add=True)
        return carry

    lax.fori_loop(0, cnt, body, 0)

    def drain(j, carry):
        pltpu.make_async_copy(ones_v, acc_s.at[cbuf.at[0, 0]], sem).wait()
        pltpu.make_async_copy(ones_v, acc_r.at[cbuf.at[0, 1]], sem).wait()
        return carry

    lax.fori_loop(0, cnt, drain, 0)
    plsc.subcore_barrier()
    pltpu.sync_copy(acc_s.at[pl.ds(base_r, ROWS_PER_TILE)], tmp)
    pltpu.sync_copy(tmp, out_hbm.at[cid, 0, pl.ds(base_r, ROWS_PER_TILE)])
    pltpu.sync_copy(acc_r.at[pl.ds(base_r, ROWS_PER_TILE)], tmp)
    pltpu.sync_copy(tmp, out_hbm.at[cid, 1, pl.ds(base_r, ROWS_PER_TILE)])


@functools.partial(
    pl.kernel,
    out_type=jax.ShapeDtypeStruct((SC_CORES, NPAD, D), jnp.float32),
    mesh=_sc_mesh,
    scratch_types=[
        pltpu.VMEM((WIN, 2, UNIT), jnp.int32),
        pltpu.VMEM((UNIT, D), jnp.float32),
        pltpu.VMEM((UNIT, D), jnp.float32),
        pltpu.VMEM_SHARED((NPAD, D), jnp.float32),
        pltpu.SemaphoreType.DMA,
        pltpu.SemaphoreType.DMA,
    ],
)
def _sc_edge_agg(h_hbm, comb_hbm, out_hbm, cbuf, rows0, rows1, acc,
                 sem0, sem1):
    cid = lax.axis_index("c")
    sid = lax.axis_index("s")
    wid = sid * SC_CORES + cid
    start, cnt = _unit_range(wid)

    def zrow(r, carry):
        for k in range(D // 16):
            rows0[r, pl.ds(k * 16, 16)] = jnp.zeros((16,), jnp.float32)
        return carry

    lax.fori_loop(0, UNIT, zrow, 0)
    base_r = sid * ROWS_PER_TILE
    for k in range(ROWS_PER_TILE // UNIT):
        pltpu.sync_copy(rows0, acc.at[pl.ds(base_r + k * UNIT, UNIT)])
    plsc.subcore_barrier()

    def window(w0, wcnt):
        pltpu.sync_copy(comb_hbm.at[pl.ds(w0, WIN)], cbuf)
        pltpu.async_copy(h_hbm.at[cbuf.at[0, 0]], rows0, sem0)

        def body(p, carry):
            u0 = 2 * p

            @pl.when(u0 + 1 < wcnt)
            def _():
                pltpu.async_copy(h_hbm.at[cbuf.at[u0 + 1, 0]], rows1, sem1)

            pltpu.make_async_copy(h_hbm.at[cbuf.at[u0, 0]], rows0,
                                  sem0).wait()
            pltpu.sync_copy(rows0, acc.at[cbuf.at[u0, 1]], add=True)

            @pl.when(u0 + 2 < wcnt)
            def _():
                pltpu.async_copy(h_hbm.at[cbuf.at[u0 + 2, 0]], rows0, sem0)

            @pl.when(u0 + 1 < wcnt)
            def _():
                pltpu.make_async_copy(h_hbm.at[cbuf.at[u0 + 1, 0]], rows1,
                                      sem1).wait()
                pltpu.sync_copy(rows1, acc.at[cbuf.at[u0 + 1, 1]], add=True)

            return carry

        lax.fori_loop(0, (wcnt + 1) // 2, body, 0)

    window(start, jnp.int32(WIN))
    window(start + WIN, cnt - WIN)

    plsc.subcore_barrier()
    for k in range(ROWS_PER_TILE // UNIT):
        r0 = base_r + k * UNIT
        pltpu.sync_copy(acc.at[pl.ds(r0, UNIT)], rows0)
        pltpu.sync_copy(rows0, out_hbm.at[cid, pl.ds(r0, UNIT)])


def kernel(nodes, senders, receivers, n_node, W_embed, b_embed, W_mlp, b_mlp,
           ln_scale, ln_bias, W_dec, b_dec):
    comb = jnp.stack(
        [senders.reshape(NUNITS, UNIT), receivers.reshape(NUNITS, UNIT)],
        axis=1,
    )
    comb = jnp.pad(comb, ((0, 48), (0, 0), (0, 0)))
    degp = _sc_degrees(comb)
    inv = _tc_inv(degp)
    inv_s_col = inv[0].reshape(NPAD, 1)
    inv_r_col = inv[1].reshape(NPAD, 1)
    x, h = _tc_embed_mlp(nodes, W_embed, b_embed, W_mlp[0], b_mlp[0],
                         inv_s_col)
    aggp = _sc_edge_agg(h, comb)
    x, h = _tc_post_mlp(aggp, h, x, inv_r_col, ln_scale[0], ln_bias[0],
                        W_mlp[1], b_mlp[1], inv_s_col)
    aggp = _sc_edge_agg(h, comb)
    return _tc_post_pool(aggp, h, x, inv_r_col, ln_scale[1], ln_bias[1],
                         n_node, W_dec, b_dec)
